# baseline (device time: 48233 ns/iter reference)
import jax
import jax.numpy as jnp
from jax import lax
from jax.experimental import pallas as pl
from jax.experimental.pallas import tpu as pltpu

N_DEV = 8
HQ = 8
DH = 128
DMODEL = HQ * DH
SQ = 256
SCALE = 0.08838834764831843

N_GRP = 8
HPG = HQ // N_GRP
GW = HPG * DH
L_OFF = GW
PAYLOAD_W = GW + HPG
PACK_W = GW + 128
N_ROUNDS = 3


def kernel(x, Wq, Wo, K_ext, V_ext):
    skv = K_ext.shape[1]
    x2 = x.reshape(SQ, DMODEL)
    k2 = K_ext.reshape(skv, HQ, DH)
    v2 = V_ext.reshape(skv, HQ, DH)

    def body(x_ref, wq_ref, wo_ref, k_ref, v_ref, out_ref,
             kbuf, vbuf, q_ref, pack_ref, recv_ref,
             kv_sems, send_sems, recv_sems):
        my = lax.axis_index("i")

        def fetch(h, slot):
            pltpu.make_async_copy(
                k_ref.at[:, h, :], kbuf.at[slot], kv_sems.at[slot, 0]
            ).start()
            pltpu.make_async_copy(
                v_ref.at[:, h, :], vbuf.at[slot], kv_sems.at[slot, 1]
            ).start()

        def fetch_wait(h, slot):
            pltpu.make_async_copy(
                k_ref.at[:, h, :], kbuf.at[slot], kv_sems.at[slot, 0]
            ).wait()
            pltpu.make_async_copy(
                v_ref.at[:, h, :], vbuf.at[slot], kv_sems.at[slot, 1]
            ).wait()

        def grp_rdma(g, r):
            return pltpu.make_async_remote_copy(
                src_ref=pack_ref.at[g],
                dst_ref=recv_ref.at[g, r],
                send_sem=send_sems.at[g, r],
                recv_sem=recv_sems.at[g, r],
                device_id=(my ^ (1 << r),),
                device_id_type=pl.DeviceIdType.MESH,
            )

        barrier_sem = pltpu.get_barrier_semaphore()
        for r in range(N_ROUNDS):
            pl.semaphore_signal(barrier_sem, inc=1,
                                device_id=(my ^ (1 << r),),
                                device_id_type=pl.DeviceIdType.MESH)
        pl.semaphore_wait(barrier_sem, N_ROUNDS)

        fetch(0, 0)
        q_ref[:, :] = jnp.dot(x_ref[:, :], wq_ref[:, :],
                              preferred_element_type=jnp.float32) * SCALE
        for h in range(HQ):
            slot = h % 2
            if h + 1 < HQ:
                fetch(h + 1, (h + 1) % 2)
            fetch_wait(h, slot)
            qh = q_ref[:, h * DH:(h + 1) * DH]
            s = lax.dot_general(qh, kbuf[slot], (((1,), (1,)), ((), ())),
                                preferred_element_type=jnp.float32)
            p = jnp.exp(s)
            l = jnp.sum(p, axis=1, keepdims=True)
            o = jnp.dot(p, vbuf[slot], preferred_element_type=jnp.float32)
            g, i = divmod(h, HPG)
            pack_ref[g, :, i * DH:(i + 1) * DH] = o.astype(jnp.bfloat16)
            pack_ref[g, :, L_OFF + i:L_OFF + i + 1] = l.astype(jnp.bfloat16)
            if i == HPG - 1:
                grp_rdma(g, 0).start()

        for r in range(N_ROUNDS):
            for g in range(N_GRP):
                grp_rdma(g, r).wait()
                pack_ref[g, :, :PAYLOAD_W] = (
                    pack_ref[g, :, :PAYLOAD_W]
                    + recv_ref[g, r, :, :PAYLOAD_W])
                if r + 1 < N_ROUNDS:
                    grp_rdma(g, r + 1).start()

        for h in range(HQ):
            g, i = divmod(h, HPG)
            q_ref[:, h * DH:(h + 1) * DH] = (
                pack_ref[g, :, i * DH:(i + 1) * DH].astype(jnp.float32)
                / pack_ref[g, :, L_OFF + i:L_OFF + i + 1]
                .astype(jnp.float32))
        out_ref[:, :] = jnp.dot(q_ref[:, :], wo_ref[:, :],
                                preferred_element_type=jnp.float32)

    out = pl.pallas_call(
        body,
        out_shape=jax.ShapeDtypeStruct((SQ, DMODEL), jnp.float32),
        in_specs=[
            pl.BlockSpec(memory_space=pltpu.VMEM),
            pl.BlockSpec(memory_space=pltpu.VMEM),
            pl.BlockSpec(memory_space=pltpu.VMEM),
            pl.BlockSpec(memory_space=pl.ANY),
            pl.BlockSpec(memory_space=pl.ANY),
        ],
        out_specs=pl.BlockSpec(memory_space=pltpu.VMEM),
        scratch_shapes=[
            pltpu.VMEM((2, skv, DH), jnp.float32),
            pltpu.VMEM((2, skv, DH), jnp.float32),
            pltpu.VMEM((SQ, DMODEL), jnp.float32),
            pltpu.VMEM((N_GRP, SQ, PACK_W), jnp.bfloat16),
            pltpu.VMEM((N_GRP, N_ROUNDS, SQ, PACK_W), jnp.bfloat16),
            pltpu.SemaphoreType.DMA((2, 2)),
            pltpu.SemaphoreType.DMA((N_GRP, N_ROUNDS)),
            pltpu.SemaphoreType.DMA((N_GRP, N_ROUNDS)),
        ],
        compiler_params=pltpu.CompilerParams(
            vmem_limit_bytes=100 * 1024 * 1024,
            collective_id=0,
        ),
    )(x2, Wq, Wo, k2, v2)
    return out.reshape(1, SQ, DMODEL)


# device time: 44492 ns/iter; 1.0841x vs baseline; 1.0841x over previous
import jax
import jax.numpy as jnp
from jax import lax
from jax.experimental import pallas as pl
from jax.experimental.pallas import tpu as pltpu

N_DEV = 8
HQ = 8
DH = 128
DMODEL = HQ * DH
SQ = 256
SCALE = 0.08838834764831843

N_GRP = 4
HPG = HQ // N_GRP
GW = HPG * DH
L_OFF = GW
PAYLOAD_W = GW + HPG
PACK_W = GW + 128
N_ROUNDS = 3


def kernel(x, Wq, Wo, K_ext, V_ext):
    skv = K_ext.shape[1]
    x2 = x.reshape(SQ, DMODEL)
    k2 = K_ext.reshape(skv, HQ, DH)
    v2 = V_ext.reshape(skv, HQ, DH)

    def body(x_ref, wq_ref, wo_ref, k_ref, v_ref, out_ref,
             kbuf, vbuf, q_ref, pack_ref, recv_ref,
             kv_sems, send_sems, recv_sems):
        my = lax.axis_index("i")

        def fetch(h, slot):
            pltpu.make_async_copy(
                k_ref.at[:, h, :], kbuf.at[slot], kv_sems.at[slot, 0]
            ).start()
            pltpu.make_async_copy(
                v_ref.at[:, h, :], vbuf.at[slot], kv_sems.at[slot, 1]
            ).start()

        def fetch_wait(h, slot):
            pltpu.make_async_copy(
                k_ref.at[:, h, :], kbuf.at[slot], kv_sems.at[slot, 0]
            ).wait()
            pltpu.make_async_copy(
                v_ref.at[:, h, :], vbuf.at[slot], kv_sems.at[slot, 1]
            ).wait()

        def grp_bit(g, r):
            return (1 << r) if g % 2 == 0 else (1 << (N_ROUNDS - 1 - r))

        def grp_rdma(g, r):
            return pltpu.make_async_remote_copy(
                src_ref=pack_ref.at[g],
                dst_ref=recv_ref.at[g, r],
                send_sem=send_sems.at[g, r],
                recv_sem=recv_sems.at[g, r],
                device_id=(my ^ grp_bit(g, r),),
                device_id_type=pl.DeviceIdType.MESH,
            )

        barrier_sem = pltpu.get_barrier_semaphore()
        for r in range(N_ROUNDS):
            pl.semaphore_signal(barrier_sem, inc=1,
                                device_id=(my ^ (1 << r),),
                                device_id_type=pl.DeviceIdType.MESH)
        pl.semaphore_wait(barrier_sem, N_ROUNDS)

        fetch(0, 0)
        q_ref[:, :] = jnp.dot(x_ref[:, :], wq_ref[:, :],
                              preferred_element_type=jnp.float32) * SCALE
        for h in range(HQ):
            slot = h % 2
            if h + 1 < HQ:
                fetch(h + 1, (h + 1) % 2)
            fetch_wait(h, slot)
            qh = q_ref[:, h * DH:(h + 1) * DH]
            s = lax.dot_general(qh, kbuf[slot], (((1,), (1,)), ((), ())),
                                preferred_element_type=jnp.float32)
            p = jnp.exp(s)
            l = jnp.sum(p, axis=1, keepdims=True)
            o = jnp.dot(p, vbuf[slot], preferred_element_type=jnp.float32)
            g, i = divmod(h, HPG)
            pack_ref[g, :, i * DH:(i + 1) * DH] = o.astype(jnp.bfloat16)
            pack_ref[g, :, L_OFF + i:L_OFF + i + 1] = l.astype(jnp.bfloat16)
            if i == HPG - 1:
                grp_rdma(g, 0).start()

        for r in range(N_ROUNDS):
            for g in range(N_GRP):
                grp_rdma(g, r).wait()
                pack_ref[g, :, :PAYLOAD_W] = (
                    pack_ref[g, :, :PAYLOAD_W]
                    + recv_ref[g, r, :, :PAYLOAD_W])
                if r + 1 < N_ROUNDS:
                    grp_rdma(g, r + 1).start()

        for h in range(HQ):
            g, i = divmod(h, HPG)
            q_ref[:, h * DH:(h + 1) * DH] = (
                pack_ref[g, :, i * DH:(i + 1) * DH].astype(jnp.float32)
                / pack_ref[g, :, L_OFF + i:L_OFF + i + 1]
                .astype(jnp.float32))
        out_ref[:, :] = jnp.dot(q_ref[:, :], wo_ref[:, :],
                                preferred_element_type=jnp.float32)

    out = pl.pallas_call(
        body,
        out_shape=jax.ShapeDtypeStruct((SQ, DMODEL), jnp.float32),
        in_specs=[
            pl.BlockSpec(memory_space=pltpu.VMEM),
            pl.BlockSpec(memory_space=pltpu.VMEM),
            pl.BlockSpec(memory_space=pltpu.VMEM),
            pl.BlockSpec(memory_space=pl.ANY),
            pl.BlockSpec(memory_space=pl.ANY),
        ],
        out_specs=pl.BlockSpec(memory_space=pltpu.VMEM),
        scratch_shapes=[
            pltpu.VMEM((2, skv, DH), jnp.float32),
            pltpu.VMEM((2, skv, DH), jnp.float32),
            pltpu.VMEM((SQ, DMODEL), jnp.float32),
            pltpu.VMEM((N_GRP, SQ, PACK_W), jnp.bfloat16),
            pltpu.VMEM((N_GRP, N_ROUNDS, SQ, PACK_W), jnp.bfloat16),
            pltpu.SemaphoreType.DMA((2, 2)),
            pltpu.SemaphoreType.DMA((N_GRP, N_ROUNDS)),
            pltpu.SemaphoreType.DMA((N_GRP, N_ROUNDS)),
        ],
        compiler_params=pltpu.CompilerParams(
            vmem_limit_bytes=100 * 1024 * 1024,
            collective_id=0,
        ),
    )(x2, Wq, Wo, k2, v2)
    return out.reshape(1, SQ, DMODEL)


# device time: 43632 ns/iter; 1.1055x vs baseline; 1.0197x over previous
import jax
import jax.numpy as jnp
from jax import lax
from jax.experimental import pallas as pl
from jax.experimental.pallas import tpu as pltpu

N_DEV = 8
HQ = 8
DH = 128
DMODEL = HQ * DH
SQ = 256
SCALE = 0.08838834764831843

N_GRP = 4
HPG = HQ // N_GRP
GW = HPG * DH
L_OFF = GW
PAYLOAD_W = GW + HPG
PACK_W = GW + 128
N_ROUNDS = 3


def kernel(x, Wq, Wo, K_ext, V_ext):
    skv = K_ext.shape[1]
    x2 = x.reshape(SQ, DMODEL)
    k2 = K_ext.reshape(skv, HQ, DH)
    v2 = V_ext.reshape(skv, HQ, DH)

    def body(x_ref, wq_ref, wo_ref, k_ref, v_ref, out_ref,
             kbuf, vbuf, q_ref, pack_ref, recv_ref,
             kv_sems, send_sems, recv_sems):
        my = lax.axis_index("i")

        def fetch(h, slot):
            pltpu.make_async_copy(
                k_ref.at[:, h, :], kbuf.at[slot], kv_sems.at[slot, 0]
            ).start()
            pltpu.make_async_copy(
                v_ref.at[:, h, :], vbuf.at[slot], kv_sems.at[slot, 1]
            ).start()

        def fetch_wait(h, slot):
            pltpu.make_async_copy(
                k_ref.at[:, h, :], kbuf.at[slot], kv_sems.at[slot, 0]
            ).wait()
            pltpu.make_async_copy(
                v_ref.at[:, h, :], vbuf.at[slot], kv_sems.at[slot, 1]
            ).wait()

        def grp_bit(g, r):
            return (1 << r) if g % 2 == 0 else (1 << (N_ROUNDS - 1 - r))

        def grp_rdma(g, r):
            return pltpu.make_async_remote_copy(
                src_ref=pack_ref.at[g],
                dst_ref=recv_ref.at[g, r],
                send_sem=send_sems.at[g, r],
                recv_sem=recv_sems.at[g, r],
                device_id=(my ^ grp_bit(g, r),),
                device_id_type=pl.DeviceIdType.MESH,
            )

        barrier_sem = pltpu.get_barrier_semaphore()
        for r in range(N_ROUNDS):
            pl.semaphore_signal(barrier_sem, inc=1,
                                device_id=(my ^ (1 << r),),
                                device_id_type=pl.DeviceIdType.MESH)
        pl.semaphore_wait(barrier_sem, N_ROUNDS)

        fetch(0, 0)
        q_ref[:, :] = jnp.dot(x_ref[:, :], wq_ref[:, :],
                              preferred_element_type=jnp.float32) * SCALE
        for h in range(HQ):
            slot = h % 2
            if h + 1 < HQ:
                fetch(h + 1, (h + 1) % 2)
            fetch_wait(h, slot)
            qh = q_ref[:, h * DH:(h + 1) * DH]
            s = lax.dot_general(qh, kbuf[slot], (((1,), (1,)), ((), ())),
                                preferred_element_type=jnp.float32)
            p = jnp.exp(s)
            l = jnp.sum(p, axis=1, keepdims=True)
            o = jnp.dot(p, vbuf[slot], preferred_element_type=jnp.float32)
            g, i = divmod(h, HPG)
            pack_ref[g, :, i * DH:(i + 1) * DH] = o.astype(jnp.bfloat16)
            pack_ref[g, :, L_OFF + i:L_OFF + i + 1] = l.astype(jnp.bfloat16)
            if i == HPG - 1:
                grp_rdma(g, 0).start()

        for r in range(N_ROUNDS):
            for g in range(N_GRP):
                grp_rdma(g, r).wait()
                pack_ref[g, :, :PAYLOAD_W] = (
                    pack_ref[g, :, :PAYLOAD_W]
                    + recv_ref[g, r, :, :PAYLOAD_W])
                if r + 1 < N_ROUNDS:
                    grp_rdma(g, r + 1).start()
                else:
                    for i in range(HPG):
                        h = g * HPG + i
                        q_ref[:, h * DH:(h + 1) * DH] = (
                            pack_ref[g, :, i * DH:(i + 1) * DH]
                            .astype(jnp.float32)
                            / pack_ref[g, :, L_OFF + i:L_OFF + i + 1]
                            .astype(jnp.float32))
                    part = jnp.dot(
                        q_ref[:, g * GW:(g + 1) * GW],
                        wo_ref[g * GW:(g + 1) * GW, :],
                        preferred_element_type=jnp.float32)
                    if g == 0:
                        out_ref[:, :] = part
                    else:
                        out_ref[:, :] = out_ref[:, :] + part

    out = pl.pallas_call(
        body,
        out_shape=jax.ShapeDtypeStruct((SQ, DMODEL), jnp.float32),
        in_specs=[
            pl.BlockSpec(memory_space=pltpu.VMEM),
            pl.BlockSpec(memory_space=pltpu.VMEM),
            pl.BlockSpec(memory_space=pltpu.VMEM),
            pl.BlockSpec(memory_space=pl.ANY),
            pl.BlockSpec(memory_space=pl.ANY),
        ],
        out_specs=pl.BlockSpec(memory_space=pltpu.VMEM),
        scratch_shapes=[
            pltpu.VMEM((2, skv, DH), jnp.float32),
            pltpu.VMEM((2, skv, DH), jnp.float32),
            pltpu.VMEM((SQ, DMODEL), jnp.float32),
            pltpu.VMEM((N_GRP, SQ, PACK_W), jnp.bfloat16),
            pltpu.VMEM((N_GRP, N_ROUNDS, SQ, PACK_W), jnp.bfloat16),
            pltpu.SemaphoreType.DMA((2, 2)),
            pltpu.SemaphoreType.DMA((N_GRP, N_ROUNDS)),
            pltpu.SemaphoreType.DMA((N_GRP, N_ROUNDS)),
        ],
        compiler_params=pltpu.CompilerParams(
            vmem_limit_bytes=100 * 1024 * 1024,
            collective_id=0,
        ),
    )(x2, Wq, Wo, k2, v2)
    return out.reshape(1, SQ, DMODEL)
